# baseline (speedup 1.0000x reference)
import functools

import jax
import jax.numpy as jnp
from jax import lax
from jax.experimental import pallas as pl
from jax.experimental.pallas import tpu as pltpu
from jax.experimental.pallas import tpu_sc as plsc

B = 2048
SEG = 1024
LAT = 64
K = 512
EMB = 64

BM = 1024

_NT = (((1,), (1,)), ((), ()))
_NN = (((1,), (0,)), ((), ()))


def _dense(h, w_ref, b_ref, relu=True):
    o = lax.dot_general(h, w_ref[...], _NT,
                        preferred_element_type=jnp.float32) + b_ref[...]
    return jnp.maximum(o, 0.0) if relu else o


def _enc_kernel(x_ref, w1_ref, b1_ref, w2_ref, b2_ref, w3_ref, b3_ref,
                w4_ref, b4_ref, cbt_ref, z_ref, idx_ref):
    f32 = jnp.float32
    h = _dense(x_ref[...], w1_ref, b1_ref)
    h = _dense(h, w2_ref, b2_ref)
    h = _dense(h, w3_ref, b3_ref)
    z = _dense(h, w4_ref, b4_ref, relu=False)
    z_ref[...] = z

    cbt = cbt_ref[...]

    def split3(v):
        b = lax.bitcast_convert_type(v, jnp.uint32)
        v0 = lax.bitcast_convert_type(b & jnp.uint32(0xFFFF0000), f32)
        r = v - v0
        rb = lax.bitcast_convert_type(r, jnp.uint32)
        v1 = lax.bitcast_convert_type(rb & jnp.uint32(0xFFFF0000), f32)
        return v0, v1, r - v1

    z0, z1, z2 = split3(z)
    c0, c1, c2 = split3(cbt)
    zs = jnp.concatenate([z0, z0, z1, z0, z2, z1], axis=1)
    cs = jnp.concatenate([c0, c1, c0, c2, c0, c1], axis=0)
    scores = lax.dot_general(zs, cs, _NN, preferred_element_type=f32)
    cn2 = jnp.sum(cbt * cbt, axis=0, keepdims=True)
    dist = cn2 - 2.0 * scores
    minval = jnp.min(dist, axis=1, keepdims=True)
    iota = lax.broadcasted_iota(jnp.int32, (BM, K), 1)
    idx_ref[...] = jnp.min(jnp.where(dist == minval, iota, K), axis=1,
                           keepdims=True)


def _dec_kernel(zq_ref, dw1_ref, db1_ref, dw2_ref, db2_ref, dw3_ref, db3_ref,
                dw4_ref, db4_ref, xr_ref):
    h = _dense(zq_ref[...], dw1_ref, db1_ref)
    h = _dense(h, dw2_ref, db2_ref)
    h = _dense(h, dw3_ref, db3_ref)
    xr_ref[...] = _dense(h, dw4_ref, db4_ref, relu=False)


def _sc_gather(table, idx):
    gw = 128
    table = jnp.pad(table, ((0, 0), (0, gw - EMB)))
    info = plsc.get_sparse_core_info()
    nw = info.num_cores * info.num_subcores
    b_per_w = B // nw
    mesh = plsc.VectorSubcoreMesh(core_axis_name="c", subcore_axis_name="s")

    @functools.partial(
        pl.kernel, mesh=mesh,
        out_type=jax.ShapeDtypeStruct((B, gw), jnp.float32),
        scratch_types=[
            pltpu.VMEM((b_per_w,), jnp.int32),
            pltpu.VMEM((b_per_w, gw), jnp.float32),
            pltpu.SemaphoreType.DMA,
        ],
    )
    def gather_k(table_hbm, idx_hbm, out_hbm, idx_v, rows_v, sem):
        wid = lax.axis_index("s") * info.num_cores + lax.axis_index("c")
        base = wid * b_per_w
        pltpu.sync_copy(idx_hbm.at[pl.ds(base, b_per_w)], idx_v)
        pltpu.async_copy(table_hbm.at[idx_v], rows_v, sem).wait()
        pltpu.sync_copy(rows_v, out_hbm.at[pl.ds(base, b_per_w)])

    return gather_k(table, idx)[:, :EMB]


@jax.jit
def kernel(x, enc_w1, enc_b1, enc_w2, enc_b2, enc_w3, enc_b3, enc_w4, enc_b4,
           codebook, dec_w1, dec_b1, dec_w2, dec_b2, dec_w3, dec_b3, dec_w4,
           dec_b4):
    def full(a):
        return pl.BlockSpec(a.shape, lambda i: (0,) * a.ndim)

    def rowblk(cols):
        return pl.BlockSpec((BM, cols), lambda i: (i, 0))

    eb = [b.reshape(1, -1) for b in (enc_b1, enc_b2, enc_b3, enc_b4)]
    db = [b.reshape(1, -1) for b in (dec_b1, dec_b2, dec_b3, dec_b4)]
    cbt = codebook.T

    z, idx = pl.pallas_call(
        _enc_kernel,
        grid=(B // BM,),
        in_specs=[rowblk(SEG),
                  full(enc_w1), full(eb[0]), full(enc_w2), full(eb[1]),
                  full(enc_w3), full(eb[2]), full(enc_w4), full(eb[3]),
                  full(cbt)],
        out_specs=(rowblk(LAT), rowblk(1)),
        out_shape=(jax.ShapeDtypeStruct((B, LAT), jnp.float32),
                   jax.ShapeDtypeStruct((B, 1), jnp.int32)),
    )(x, enc_w1, eb[0], enc_w2, eb[1], enc_w3, eb[2], enc_w4, eb[3], cbt)

    zq = _sc_gather(codebook, idx.reshape(B))

    xr = pl.pallas_call(
        _dec_kernel,
        grid=(B // BM,),
        in_specs=[rowblk(EMB),
                  full(dec_w1), full(db[0]), full(dec_w2), full(db[1]),
                  full(dec_w3), full(db[2]), full(dec_w4), full(db[3])],
        out_specs=rowblk(SEG),
        out_shape=jax.ShapeDtypeStruct((B, SEG), jnp.float32),
    )(zq, dec_w1, db[0], dec_w2, db[1], dec_w3, db[2], dec_w4, db[3])
    return (xr, z, zq)


# speedup vs baseline: 2.3841x; 2.3841x over previous
import functools

import jax
import jax.numpy as jnp
from jax import lax
from jax.experimental import pallas as pl

B = 2048
SEG = 1024
LAT = 64
K = 512
EMB = 64

BM = 1024

_NT = (((1,), (1,)), ((), ()))
_NN = (((1,), (0,)), ((), ()))


def _vqvae_kernel(x_ref, w1_ref, b1_ref, w2_ref, b2_ref, w3_ref, b3_ref,
                  w4_ref, b4_ref, cb_ref, cbt_ref, dw1_ref, db1_ref, dw2_ref,
                  db2_ref, dw3_ref, db3_ref, dw4_ref, db4_ref,
                  xr_ref, z_ref, zq_ref):
    f32 = jnp.float32

    def dense(h, w_ref, b_ref, relu=True, precision=None):
        o = lax.dot_general(h, w_ref[...], _NT, preferred_element_type=f32,
                            precision=precision) + b_ref[...]
        return jnp.maximum(o, 0.0) if relu else o

    h = dense(x_ref[...], w1_ref, b1_ref)
    h = dense(h, w2_ref, b2_ref)
    h = dense(h, w3_ref, b3_ref)
    z = dense(h, w4_ref, b4_ref, relu=False)
    z_ref[...] = z

    cb = cb_ref[...]
    cbt = cbt_ref[...]

    def split3(v):
        b = lax.bitcast_convert_type(v, jnp.uint32)
        v0 = lax.bitcast_convert_type(b & jnp.uint32(0xFFFF0000), f32)
        r = v - v0
        rb = lax.bitcast_convert_type(r, jnp.uint32)
        v1 = lax.bitcast_convert_type(rb & jnp.uint32(0xFFFF0000), f32)
        return v0, v1, r - v1

    z0, z1, z2 = split3(z)
    c0, c1, c2 = split3(-cbt)
    ones = jnp.ones((BM, 8), f32)
    cn2 = 0.5 * jnp.sum(cbt * cbt, axis=0, keepdims=True)
    cn2rows = jnp.concatenate([cn2, jnp.zeros((7, K), f32)], axis=0)
    zs = jnp.concatenate([z0, z0, z1, z0, z2, z1, ones], axis=1)
    cs = jnp.concatenate([c0, c1, c0, c2, c0, c1, cn2rows], axis=0)
    dist = lax.dot_general(zs, cs, _NN, preferred_element_type=f32)
    idx = jnp.argmin(dist, axis=1)[:, None]
    iota = lax.broadcasted_iota(jnp.int32, (BM, K), 1)
    onehot = (iota == idx).astype(f32)
    z_q = lax.dot_general(onehot, cb, _NN, preferred_element_type=f32)
    zq_ref[...] = z_q

    h = dense(z_q, dw1_ref, db1_ref)
    h = dense(h, dw2_ref, db2_ref)
    h = dense(h, dw3_ref, db3_ref)
    xr_ref[...] = dense(h, dw4_ref, db4_ref, relu=False)


@functools.partial(jax.jit, static_argnames=())
def kernel(x, enc_w1, enc_b1, enc_w2, enc_b2, enc_w3, enc_b3, enc_w4, enc_b4,
           codebook, dec_w1, dec_b1, dec_w2, dec_b2, dec_w3, dec_b3, dec_w4,
           dec_b4):
    def full(a):
        return pl.BlockSpec(a.shape, lambda i: (0,) * a.ndim)

    def rowblk(cols):
        return pl.BlockSpec((BM, cols), lambda i: (i, 0))

    biases2d = [b.reshape(1, -1) for b in
                (enc_b1, enc_b2, enc_b3, enc_b4, dec_b1, dec_b2, dec_b3,
                 dec_b4)]
    cbt = codebook.T

    grid = (B // BM,)
    out_shape = (
        jax.ShapeDtypeStruct((B, SEG), jnp.float32),
        jax.ShapeDtypeStruct((B, LAT), jnp.float32),
        jax.ShapeDtypeStruct((B, LAT), jnp.float32),
    )
    xr, z, zq = pl.pallas_call(
        _vqvae_kernel,
        grid=grid,
        in_specs=[
            rowblk(SEG),
            full(enc_w1), full(biases2d[0]),
            full(enc_w2), full(biases2d[1]),
            full(enc_w3), full(biases2d[2]),
            full(enc_w4), full(biases2d[3]),
            full(codebook), full(cbt),
            full(dec_w1), full(biases2d[4]),
            full(dec_w2), full(biases2d[5]),
            full(dec_w3), full(biases2d[6]),
            full(dec_w4), full(biases2d[7]),
        ],
        out_specs=(rowblk(SEG), rowblk(LAT), rowblk(LAT)),
        out_shape=out_shape,
    )(x, enc_w1, biases2d[0], enc_w2, biases2d[1], enc_w3, biases2d[2],
      enc_w4, biases2d[3], codebook, cbt, dec_w1, biases2d[4], dec_w2,
      biases2d[5],
      dec_w3, biases2d[6], dec_w4, biases2d[7])
    return (xr, z, zq)


# speedup vs baseline: 2.3844x; 1.0001x over previous
import functools

import jax
import jax.numpy as jnp
from jax import lax
from jax.experimental import pallas as pl

B = 2048
SEG = 1024
LAT = 64
K = 512
EMB = 64

BM = 1024

_NT = (((1,), (1,)), ((), ()))
_NN = (((1,), (0,)), ((), ()))


def _vqvae_kernel(x_ref, w1_ref, b1_ref, w2_ref, b2_ref, w3_ref, b3_ref,
                  w4_ref, b4_ref, cb_ref, cbt_ref, dw1_ref, db1_ref, dw2_ref,
                  db2_ref, dw3_ref, db3_ref, dw4_ref, db4_ref,
                  xr_ref, z_ref, zq_ref):
    f32 = jnp.float32

    def dense(h, w_ref, b_ref, relu=True, precision=None):
        o = lax.dot_general(h, w_ref[...], _NT, preferred_element_type=f32,
                            precision=precision) + b_ref[...]
        return jnp.maximum(o, 0.0) if relu else o

    h = dense(x_ref[...], w1_ref, b1_ref)
    h = dense(h, w2_ref, b2_ref)
    h = dense(h, w3_ref, b3_ref)
    z = dense(h, w4_ref, b4_ref, relu=False)
    z_ref[...] = z

    cb = cb_ref[...]
    cbt = cbt_ref[...]

    def split3(v):
        b = lax.bitcast_convert_type(v, jnp.uint32)
        v0 = lax.bitcast_convert_type(b & jnp.uint32(0xFFFF0000), f32)
        r = v - v0
        rb = lax.bitcast_convert_type(r, jnp.uint32)
        v1 = lax.bitcast_convert_type(rb & jnp.uint32(0xFFFF0000), f32)
        return v0, v1, r - v1

    z0, z1, z2 = split3(z)
    c0, c1, c2 = split3(-cbt)
    ones = jnp.ones((BM, 8), f32)
    cn2 = 0.5 * jnp.sum(cbt * cbt, axis=0, keepdims=True)
    n0, n1, n2 = split3(cn2)
    cn2rows = jnp.concatenate([n0, n1, n2, jnp.zeros((5, K), f32)], axis=0)
    zs = jnp.concatenate([z0, z0, z1, z0, z2, z1, ones], axis=1)
    cs = jnp.concatenate([c0, c1, c0, c2, c0, c1, cn2rows], axis=0)
    dist = lax.dot_general(zs, cs, _NN, preferred_element_type=f32)
    idx = jnp.argmin(dist, axis=1)[:, None]
    iota = lax.broadcasted_iota(jnp.int32, (BM, K), 1)
    onehot = (iota == idx).astype(f32)
    z_q = lax.dot_general(onehot, cb, _NN, preferred_element_type=f32)
    zq_ref[...] = z_q

    h = dense(z_q, dw1_ref, db1_ref)
    h = dense(h, dw2_ref, db2_ref)
    h = dense(h, dw3_ref, db3_ref)
    xr_ref[...] = dense(h, dw4_ref, db4_ref, relu=False)


@functools.partial(jax.jit, static_argnames=())
def kernel(x, enc_w1, enc_b1, enc_w2, enc_b2, enc_w3, enc_b3, enc_w4, enc_b4,
           codebook, dec_w1, dec_b1, dec_w2, dec_b2, dec_w3, dec_b3, dec_w4,
           dec_b4):
    def full(a):
        return pl.BlockSpec(a.shape, lambda i: (0,) * a.ndim)

    def rowblk(cols):
        return pl.BlockSpec((BM, cols), lambda i: (i, 0))

    biases2d = [b.reshape(1, -1) for b in
                (enc_b1, enc_b2, enc_b3, enc_b4, dec_b1, dec_b2, dec_b3,
                 dec_b4)]
    cbt = codebook.T

    grid = (B // BM,)
    out_shape = (
        jax.ShapeDtypeStruct((B, SEG), jnp.float32),
        jax.ShapeDtypeStruct((B, LAT), jnp.float32),
        jax.ShapeDtypeStruct((B, LAT), jnp.float32),
    )
    xr, z, zq = pl.pallas_call(
        _vqvae_kernel,
        grid=grid,
        in_specs=[
            rowblk(SEG),
            full(enc_w1), full(biases2d[0]),
            full(enc_w2), full(biases2d[1]),
            full(enc_w3), full(biases2d[2]),
            full(enc_w4), full(biases2d[3]),
            full(codebook), full(cbt),
            full(dec_w1), full(biases2d[4]),
            full(dec_w2), full(biases2d[5]),
            full(dec_w3), full(biases2d[6]),
            full(dec_w4), full(biases2d[7]),
        ],
        out_specs=(rowblk(SEG), rowblk(LAT), rowblk(LAT)),
        out_shape=out_shape,
    )(x, enc_w1, biases2d[0], enc_w2, biases2d[1], enc_w3, biases2d[2],
      enc_w4, biases2d[3], codebook, cbt, dec_w1, biases2d[4], dec_w2,
      biases2d[5],
      dec_w3, biases2d[6], dec_w4, biases2d[7])
    return (xr, z, zq)
